# baseline (speedup 1.0000x reference)
import functools

import jax
import jax.numpy as jnp
from jax.experimental import pallas as pl

N = 10000
NPAD = 10016
D_IN = 128
HIDDEN = 64
N_CLASSES = 8


def _tc1_body(x_ref, w_ref, a_ref, h_ref, ab_ref):
    h = jnp.dot(x_ref[...], w_ref[...], preferred_element_type=jnp.float32)
    h_ref[...] = h
    ab_ref[...] = jnp.dot(h, a_ref[...], preferred_element_type=jnp.float32)


def _tc1(xpad, W1, A1):
    return pl.pallas_call(
        _tc1_body,
        out_shape=[
            jax.ShapeDtypeStruct((NPAD, HIDDEN), jnp.float32),
            jax.ShapeDtypeStruct((NPAD, 2), jnp.float32),
        ],
    )(xpad, W1, A1)


def _tc2_body(num_ref, den_ref, b_ref, w_ref, a_ref, h2_ref, ab2_ref):
    den = den_ref[...] + 1e-16
    out1 = jax.nn.relu(num_ref[...] / den + b_ref[...])
    h2 = jnp.dot(out1, w_ref[...], preferred_element_type=jnp.float32)
    h2p = jnp.concatenate(
        [h2, jnp.zeros((NPAD, 16 - N_CLASSES), jnp.float32)], axis=1
    )
    h2_ref[...] = h2p
    ab2_ref[...] = jnp.dot(h2, a_ref[...], preferred_element_type=jnp.float32)


def _tc2(num, den, b1, W2, A2):
    return pl.pallas_call(
        _tc2_body,
        out_shape=[
            jax.ShapeDtypeStruct((NPAD, 16), jnp.float32),
            jax.ShapeDtypeStruct((NPAD, 2), jnp.float32),
        ],
    )(num, den, b1[None, :], W2, A2)


def _tc3_body(num_ref, den_ref, b_ref, out_ref):
    den = den_ref[...] + 1e-16
    out_ref[...] = num_ref[...] / den + b_ref[...]


def _tc3(num, den, b2):
    return pl.pallas_call(
        _tc3_body,
        out_shape=jax.ShapeDtypeStruct((NPAD, N_CLASSES), jnp.float32),
    )(num, den, b2[None, :])


def _edge_phase_jax(h, ab, src, dst, dcols):
    e = ab[src, 0] + ab[dst, 1]
    e = jnp.where(e >= 0.0, e, 0.2 * e)
    p = jnp.exp(e)
    den = jax.ops.segment_sum(p, dst, num_segments=NPAD)
    num = jax.ops.segment_sum(h[src, :dcols] * p[:, None], dst, num_segments=NPAD)
    return num, den[:, None]


def kernel(x, edge_index, W1, a_src1, a_dst1, b1, W2, a_src2, a_dst2, b2):
    loop = jnp.arange(N, dtype=edge_index.dtype)
    src = jnp.concatenate([edge_index[0], loop])
    dst = jnp.concatenate([edge_index[1], loop])
    xpad = jnp.pad(x, ((0, NPAD - N), (0, 0)))
    A1 = jnp.stack([a_src1, a_dst1], axis=1)
    A2 = jnp.stack([a_src2, a_dst2], axis=1)

    h1, ab1 = _tc1(xpad, W1, A1)
    num1, den1 = _edge_phase_jax(h1, ab1, src, dst, HIDDEN)
    h2p, ab2 = _tc2(num1, den1, b1, W2, A2)
    num2, den2 = _edge_phase_jax(h2p, ab2, src, dst, N_CLASSES)
    out = _tc3(num2, den2, b2)
    return out[:N]


# speedup vs baseline: 146.8770x; 146.8770x over previous
import functools

import jax
import jax.numpy as jnp
from jax import lax
from jax.experimental import pallas as pl
from jax.experimental.pallas import tpu as pltpu
from jax.experimental.pallas import tpu_sc as plsc

N = 10000
NPAD = 10240
D_IN = 128
HIDDEN = 64
N_CLASSES = 8

NW = 32
C = 128
E_TOT = 640000 + N
CHUNKS_PER_W = -(-E_TOT // (NW * C))
E_PAD = NW * C * CHUNKS_PER_W
ROWS_PER_SUB = NPAD // 16


def _tc1_body(x_ref, w_ref, a_ref, h_ref, ab_ref):
    h = jnp.dot(x_ref[...], w_ref[...], preferred_element_type=jnp.float32)
    h_ref[:, :HIDDEN] = h
    h_ref[:, HIDDEN:] = jnp.zeros((NPAD, 128 - HIDDEN), jnp.float32)
    ab_ref[...] = jnp.dot(h, a_ref[...], preferred_element_type=jnp.float32)


def _tc1(xpad, W1, A1):
    return pl.pallas_call(
        _tc1_body,
        out_shape=[
            jax.ShapeDtypeStruct((NPAD, 128), jnp.float32),
            jax.ShapeDtypeStruct((NPAD, 2), jnp.float32),
        ],
    )(xpad, W1, A1)


def _tc2_body(acc_ref, den_ref, b_ref, w_ref, a_ref, h2_ref, ab2_ref):
    num = acc_ref[0] + acc_ref[1]
    den = den_ref[0, :, 0:1] + den_ref[1, :, 0:1] + 1e-16
    out1 = jax.nn.relu(num / den + b_ref[...])
    h2 = jnp.dot(out1, w_ref[...], preferred_element_type=jnp.float32)
    h2_ref[...] = jnp.concatenate(
        [h2, jnp.zeros((NPAD, 16 - N_CLASSES), jnp.float32)], axis=1
    )
    ab2_ref[...] = jnp.dot(h2, a_ref[...], preferred_element_type=jnp.float32)


def _tc2(acc, den, b1, W2, A2):
    return pl.pallas_call(
        _tc2_body,
        out_shape=[
            jax.ShapeDtypeStruct((NPAD, 16), jnp.float32),
            jax.ShapeDtypeStruct((NPAD, 2), jnp.float32),
        ],
    )(acc, den, b1[None, :], W2, A2)


def _tc3_body(acc_ref, den_ref, b_ref, out_ref):
    den = den_ref[0, :, 0:1] + den_ref[1, :, 0:1] + 1e-16
    num = acc_ref[0] + acc_ref[1]
    out_ref[...] = num / den + b_ref[...]


def _tc3(acc, den, b2pad):
    return pl.pallas_call(
        _tc3_body,
        out_shape=jax.ShapeDtypeStruct((NPAD, 16), jnp.float32),
    )(acc, den, b2pad[None, :])


def _make_sc_edge_wide():
    mesh = plsc.VectorSubcoreMesh(core_axis_name="c", subcore_axis_name="s")
    cp = pltpu.CompilerParams(needs_layout_passes=False)
    AROWS = NPAD // 2
    DROWS = NPAD // 8
    AR_SUB = AROWS // 16
    DR_SUB = DROWS // 16

    @functools.partial(
        pl.kernel,
        mesh=mesh,
        compiler_params=cp,
        out_type=[
            jax.ShapeDtypeStruct((2, AROWS, 128), jnp.float32),
            jax.ShapeDtypeStruct((2, DROWS, 128), jnp.float32),
        ],
        scratch_types=[
            pltpu.VMEM((NPAD // 128, 128), jnp.float32),
            pltpu.VMEM((NPAD // 128, 128), jnp.float32),
            pltpu.VMEM((C,), jnp.int32),
            pltpu.VMEM((1, C), jnp.int32),
            pltpu.VMEM((1, C), jnp.int32),
            pltpu.VMEM((1, C), jnp.int32),
            pltpu.VMEM((C, 128), jnp.float32),
            pltpu.VMEM((C, 128), jnp.float32),
            pltpu.VMEM((C, 128), jnp.float32),
            pltpu.VMEM((C + 16,), jnp.float32),
            pltpu.VMEM_SHARED((AROWS, 128), jnp.float32),
            pltpu.VMEM_SHARED((DROWS, 128), jnp.float32),
            pltpu.SemaphoreType.DMA,
        ],
    )
    def sc_edge(ht_hbm, as_hbm, ad_hbm, src_hbm, dst_hbm, acc_out, den_out,
                as_t, ad_t, sidx, didxr, didx2, didx8, rowbuf, outbuf, pbuf,
                pv, acc_s, den_s, sem):
        cid = lax.axis_index("c")
        sid = lax.axis_index("s")
        wid = sid * 2 + cid
        iota16 = lax.iota(jnp.int32, 16)
        fz16 = jnp.zeros((16,), jnp.float32)

        @pl.loop(0, C)
        def _zero_bufs(r):
            for j in range(8):
                outbuf[r, pl.ds(j * 16, 16)] = fz16
                pbuf[r, pl.ds(j * 16, 16)] = fz16

        @pl.loop(0, AR_SUB // 64)
        def _zero_acc(b):
            pltpu.sync_copy(
                outbuf.at[pl.ds(0, 64)],
                acc_s.at[pl.ds(sid * AR_SUB + b * 64, 64)],
            )

        pltpu.sync_copy(pbuf.at[pl.ds(0, DR_SUB)],
                        den_s.at[pl.ds(sid * DR_SUB, DR_SUB)])
        pltpu.sync_copy(as_hbm, as_t)
        pltpu.sync_copy(ad_hbm, ad_t)
        plsc.subcore_barrier()

        @pl.loop(0, CHUNKS_PER_W)
        def _chunk(t):
            off = wid * (CHUNKS_PER_W * C) + t * C
            pltpu.sync_copy(src_hbm.at[pl.ds(off, C)], sidx)
            pltpu.sync_copy(dst_hbm.at[pl.ds(off, C)], didxr.at[0])
            pltpu.async_copy(ht_hbm.at[sidx], rowbuf, sem).wait()
            for g in range(C // 16):
                sv = sidx[pl.ds(g * 16, 16)]
                dv = didxr[0, pl.ds(g * 16, 16)]
                a_s = plsc.load_gather(
                    as_t, [lax.shift_right_logical(sv, 7), sv & 127])
                a_d = plsc.load_gather(
                    ad_t, [lax.shift_right_logical(dv, 7), dv & 127])
                e = a_s + a_d
                e = jnp.where(e >= 0.0, e, e * 0.2)
                p = jnp.exp(e)
                plsc.store_scatter(pv, [g * 16 + iota16], p)
                didx2[0, pl.ds(g * 16, 16)] = lax.shift_right_logical(dv, 1)
                didx8[0, pl.ds(g * 16, 16)] = lax.shift_right_logical(dv, 3)
                plsc.store_scatter(
                    pbuf, [g * 16 + iota16, lax.shift_left(dv & 7, 4)], p)

            @pl.loop(0, C)
            def _scale(r):
                ps = pv[pl.ds(r, 16)][0]
                half = (didxr[0, pl.ds(r, 16)][0] & 1) * 64
                for j in range(HIDDEN // 16):
                    outbuf[r, pl.ds(half + j * 16, 16)] = (
                        rowbuf[r, pl.ds(j * 16, 16)] * ps
                    )
                    outbuf[r, pl.ds(64 - half + j * 16, 16)] = fz16

            pltpu.sync_copy(outbuf, acc_s.at[didx2.at[0]], add=True)
            pltpu.sync_copy(pbuf, den_s.at[didx8.at[0]], add=True)

            for g in range(C // 16):
                dv = didxr[0, pl.ds(g * 16, 16)]
                plsc.store_scatter(
                    pbuf, [g * 16 + iota16, lax.shift_left(dv & 7, 4)], fz16)

        plsc.subcore_barrier()

        @pl.loop(0, AR_SUB // 64)
        def _flush_acc(b):
            r0 = sid * AR_SUB + b * 64
            pltpu.sync_copy(acc_s.at[pl.ds(r0, 64)],
                            acc_out.at[cid].at[pl.ds(r0, 64)])

        pltpu.sync_copy(den_s.at[pl.ds(sid * DR_SUB, DR_SUB)],
                        den_out.at[cid].at[pl.ds(sid * DR_SUB, DR_SUB)])

    return sc_edge


def _make_sc_edge16():
    D = 16
    mesh = plsc.VectorSubcoreMesh(core_axis_name="c", subcore_axis_name="s")
    cp = pltpu.CompilerParams(
        needs_layout_passes=False, use_tc_tiling_on_sc=False
    )

    @functools.partial(
        pl.kernel,
        mesh=mesh,
        compiler_params=cp,
        out_type=[
            jax.ShapeDtypeStruct((2, NPAD, D), jnp.float32),
            jax.ShapeDtypeStruct((2, NPAD, 16), jnp.float32),
        ],
        scratch_types=[
            pltpu.VMEM((NPAD, 2), jnp.float32),
            pltpu.VMEM((C,), jnp.int32),
            pltpu.VMEM((1, C), jnp.int32),
            pltpu.VMEM((C, D), jnp.float32),
            pltpu.VMEM((C, 16), jnp.float32),
            pltpu.VMEM((C, D), jnp.float32),
            pltpu.VMEM_SHARED((NPAD, D), jnp.float32),
            pltpu.VMEM_SHARED((NPAD, D), jnp.float32),
            pltpu.VMEM_SHARED((NPAD, 16), jnp.float32),
            pltpu.SemaphoreType.DMA,
        ],
    )
    def sc_edge(h_hbm, ab_hbm, src_hbm, dst_hbm, acc_out, den_out,
                ab_t, sidx, didx, rowbuf, pbuf, zbuf, h_s, acc_s, den_s, sem):
        cid = lax.axis_index("c")
        sid = lax.axis_index("s")
        wid = sid * 2 + cid
        zero16 = jnp.zeros((16,), jnp.int32)
        one16 = jnp.ones((16,), jnp.int32)
        iota16 = lax.iota(jnp.int32, 16)
        fz16 = jnp.zeros((16,), jnp.float32)

        @pl.loop(0, C)
        def _zero_bufs(r):
            zbuf[r, pl.ds(0, 16)] = fz16
            pbuf[r, pl.ds(0, 16)] = fz16

        base_row = sid * ROWS_PER_SUB
        pltpu.sync_copy(
            h_hbm.at[pl.ds(base_row, ROWS_PER_SUB)],
            h_s.at[pl.ds(base_row, ROWS_PER_SUB)],
        )

        @pl.loop(0, ROWS_PER_SUB // C)
        def _zero_spmem(b):
            r0 = base_row + b * C
            pltpu.sync_copy(zbuf, acc_s.at[pl.ds(r0, C)])
            pltpu.sync_copy(pbuf, den_s.at[pl.ds(r0, C)])

        pltpu.sync_copy(ab_hbm, ab_t)
        plsc.subcore_barrier()

        @pl.loop(0, CHUNKS_PER_W)
        def _chunk(t):
            off = wid * (CHUNKS_PER_W * C) + t * C
            pltpu.sync_copy(src_hbm.at[pl.ds(off, C)], sidx)
            pltpu.sync_copy(dst_hbm.at[pl.ds(off, C)], didx.at[0])
            pltpu.async_copy(h_s.at[sidx], rowbuf, sem).wait()
            for g in range(C // 16):
                sv = sidx[pl.ds(g * 16, 16)]
                dv = didx[0, pl.ds(g * 16, 16)]
                a_s = plsc.load_gather(ab_t, [sv, zero16])
                a_d = plsc.load_gather(ab_t, [dv, one16])
                e = a_s + a_d
                e = jnp.where(e >= 0.0, e, e * 0.2)
                p = jnp.exp(e)
                plsc.store_scatter(pbuf, [g * 16 + iota16, zero16], p)

            @pl.loop(0, C)
            def _scale(r):
                ps = pbuf[r, pl.ds(0, 16)][0]
                rowbuf[r, pl.ds(0, 16)] = rowbuf[r, pl.ds(0, 16)] * ps

            pltpu.sync_copy(rowbuf, acc_s.at[didx.at[0]], add=True)
            pltpu.sync_copy(pbuf, den_s.at[didx.at[0]], add=True)

        plsc.subcore_barrier()

        @pl.loop(0, ROWS_PER_SUB // C)
        def _flush(b):
            r0 = base_row + b * C
            pltpu.sync_copy(acc_s.at[pl.ds(r0, C)], acc_out.at[cid].at[pl.ds(r0, C)])
            pltpu.sync_copy(den_s.at[pl.ds(r0, C)], den_out.at[cid].at[pl.ds(r0, C)])

    return sc_edge


_sc_edge_l1 = _make_sc_edge_wide()
_sc_edge_l2 = _make_sc_edge16()


def kernel(x, edge_index, W1, a_src1, a_dst1, b1, W2, a_src2, a_dst2, b2):
    loop = jnp.arange(N, dtype=edge_index.dtype)
    src = jnp.concatenate([edge_index[0], loop])
    dst = jnp.concatenate([edge_index[1], loop])
    src = jnp.pad(src, (0, E_PAD - E_TOT), constant_values=N)
    dst = jnp.pad(dst, (0, E_PAD - E_TOT), constant_values=N)
    xpad = jnp.pad(x, ((0, NPAD - N), (0, 0)))
    A1 = jnp.stack([a_src1, a_dst1], axis=1)
    A2 = jnp.stack([a_src2, a_dst2], axis=1)
    b2pad = jnp.pad(b2, (0, 16 - N_CLASSES))

    ht1, ab1 = _tc1(xpad, W1, A1)
    as1 = ab1[:, 0].reshape(NPAD // 128, 128)
    ad1 = ab1[:, 1].reshape(NPAD // 128, 128)
    acc1p, den1p = _sc_edge_l1(ht1, as1, ad1, src, dst)
    acc1 = acc1p.reshape(2, NPAD, HIDDEN)
    den1 = den1p.reshape(2, NPAD, 16)
    h2p, ab2 = _tc2(acc1, den1, b1, W2, A2)
    acc2, den2 = _sc_edge_l2(h2p, ab2, src, dst)
    out = _tc3(acc2, den2, b2pad)
    return out[:N, :N_CLASSES]


# speedup vs baseline: 160.4330x; 1.0923x over previous
import functools

import jax
import jax.numpy as jnp
from jax import lax
from jax.experimental import pallas as pl
from jax.experimental.pallas import tpu as pltpu
from jax.experimental.pallas import tpu_sc as plsc

N = 10000
NPAD = 10240
D_IN = 128
HIDDEN = 64
N_CLASSES = 8

NW = 32
C = 128
E_TOT = 640000 + N
CHUNKS_PER_W = 160
E_PAD = NW * C * CHUNKS_PER_W
ROWS_PER_SUB = NPAD // 16


def _tc1_body(x_ref, w_ref, a_ref, h_ref, ab_ref):
    h = jnp.dot(x_ref[...], w_ref[...], preferred_element_type=jnp.float32)
    h_ref[:, :HIDDEN] = h
    h_ref[:, HIDDEN:] = jnp.zeros((NPAD, 128 - HIDDEN), jnp.float32)
    ab_ref[...] = jnp.dot(h, a_ref[...], preferred_element_type=jnp.float32)


def _tc1(xpad, W1, A1):
    return pl.pallas_call(
        _tc1_body,
        out_shape=[
            jax.ShapeDtypeStruct((NPAD, 128), jnp.float32),
            jax.ShapeDtypeStruct((NPAD, 2), jnp.float32),
        ],
    )(xpad, W1, A1)


def _tc2_body(acc_ref, den_ref, b_ref, w_ref, a_ref, h2_ref, ab2_ref):
    num = acc_ref[0] + acc_ref[1]
    den = den_ref[0, :, 0:1] + den_ref[1, :, 0:1] + 1e-16
    out1 = jax.nn.relu(num / den + b_ref[...])
    h2 = jnp.dot(out1, w_ref[...], preferred_element_type=jnp.float32)
    h2_ref[...] = jnp.concatenate(
        [h2, jnp.zeros((NPAD, 16 - N_CLASSES), jnp.float32)], axis=1
    )
    ab2_ref[...] = jnp.dot(h2, a_ref[...], preferred_element_type=jnp.float32)


def _tc2(acc, den, b1, W2, A2):
    return pl.pallas_call(
        _tc2_body,
        out_shape=[
            jax.ShapeDtypeStruct((NPAD, 16), jnp.float32),
            jax.ShapeDtypeStruct((NPAD, 2), jnp.float32),
        ],
    )(acc, den, b1[None, :], W2, A2)


def _tc3_body(acc_ref, den_ref, b_ref, out_ref):
    den = den_ref[0, :, 0:1] + den_ref[1, :, 0:1] + 1e-16
    num = acc_ref[0] + acc_ref[1]
    out_ref[...] = num / den + b_ref[...]


def _tc3(acc, den, b2pad):
    return pl.pallas_call(
        _tc3_body,
        out_shape=jax.ShapeDtypeStruct((NPAD, 16), jnp.float32),
    )(acc, den, b2pad[None, :])


def _make_sc_edge_wide():
    mesh = plsc.VectorSubcoreMesh(core_axis_name="c", subcore_axis_name="s")
    cp = pltpu.CompilerParams(needs_layout_passes=False)
    AROWS = NPAD // 2
    DROWS = NPAD // 128
    AR_SUB = AROWS // 16
    DR_SUB = DROWS // 16

    @functools.partial(
        pl.kernel,
        mesh=mesh,
        compiler_params=cp,
        out_type=[
            jax.ShapeDtypeStruct((2, AROWS, 128), jnp.float32),
            jax.ShapeDtypeStruct((2, DROWS, 128), jnp.float32),
        ],
        scratch_types=[
            pltpu.VMEM((NPAD // 128, 128), jnp.float32),
            pltpu.VMEM((NPAD // 128, 128), jnp.float32),
            pltpu.VMEM((2, C), jnp.int32),
            pltpu.VMEM((2, C), jnp.int32),
            pltpu.VMEM((1, C), jnp.int32),
            pltpu.VMEM((1, C), jnp.int32),
            pltpu.VMEM((C + 16,), jnp.int32),
            pltpu.VMEM((2, C, 128), jnp.float32),
            pltpu.VMEM((C, 128), jnp.float32),
            pltpu.VMEM((C, 128), jnp.float32),
            pltpu.VMEM((C + 16,), jnp.float32),
            pltpu.VMEM_SHARED((AROWS, 128), jnp.float32),
            pltpu.VMEM_SHARED((DROWS, 128), jnp.float32),
            pltpu.SemaphoreType.DMA,
            pltpu.SemaphoreType.DMA,
        ],
    )
    def sc_edge(ht_hbm, as_hbm, ad_hbm, src_hbm, dst_hbm, acc_out, den_out,
                as_t, ad_t, sidx, didxr, didx2, didx8, ph, rowbuf, outbuf,
                pbuf, pv, acc_s, den_s, sem0, sem1):
        cid = lax.axis_index("c")
        sid = lax.axis_index("s")
        wid = sid * 2 + cid
        iota16 = lax.iota(jnp.int32, 16)
        fz16 = jnp.zeros((16,), jnp.float32)
        sems = (sem0, sem1)
        base = wid * (CHUNKS_PER_W * C)

        @pl.loop(0, C)
        def _zero_bufs(r):
            for j in range(8):
                outbuf[r, pl.ds(j * 16, 16)] = fz16
                pbuf[r, pl.ds(j * 16, 16)] = fz16

        @pl.loop(0, AR_SUB // 64)
        def _zero_acc(b):
            pltpu.sync_copy(
                outbuf.at[pl.ds(0, 64)],
                acc_s.at[pl.ds(sid * AR_SUB + b * 64, 64)],
            )

        @pl.when(sid < DROWS // 8)
        def _zero_den():
            pltpu.sync_copy(pbuf.at[pl.ds(0, 8)],
                            den_s.at[pl.ds(sid * 8, 8)])

        pltpu.sync_copy(as_hbm, as_t)
        pltpu.sync_copy(ad_hbm, ad_t)
        plsc.subcore_barrier()

        def fetch(t, b):
            off = base + t * C
            pltpu.sync_copy(src_hbm.at[pl.ds(off, C)], sidx.at[b])
            pltpu.sync_copy(dst_hbm.at[pl.ds(off, C)], didxr.at[b])
            pltpu.async_copy(ht_hbm.at[sidx.at[b]], rowbuf.at[b], sems[b])

        def process(b):
            pltpu.make_async_copy(
                ht_hbm.at[sidx.at[b]], rowbuf.at[b], sems[b]).wait()
            for g in range(C // 16):
                sv = sidx[b, pl.ds(g * 16, 16)]
                dv = didxr[b, pl.ds(g * 16, 16)]
                a_s = plsc.load_gather(
                    as_t, [lax.shift_right_logical(sv, 7), sv & 127])
                a_d = plsc.load_gather(
                    ad_t, [lax.shift_right_logical(dv, 7), dv & 127])
                e = a_s + a_d
                e = jnp.where(e >= 0.0, e, e * 0.2)
                p = jnp.exp(e)
                plsc.store_scatter(pv, [g * 16 + iota16], p)
                didx2[0, pl.ds(g * 16, 16)] = lax.shift_right_logical(dv, 1)
                didx8[0, pl.ds(g * 16, 16)] = lax.shift_right_logical(dv, 7)
                plsc.store_scatter(ph, [g * 16 + iota16],
                                   lax.shift_left(dv & 1, 6))
                plsc.store_scatter(pbuf, [g * 16 + iota16, dv & 127], p)

            @pl.loop(0, C)
            def _scale(r):
                ps = pv[pl.ds(r, 16)][0]
                half = ph[pl.ds(r, 16)][0]
                for j in range(HIDDEN // 16):
                    outbuf[r, pl.ds(half + j * 16, 16)] = (
                        rowbuf[b, r, pl.ds(j * 16, 16)] * ps
                    )
                    outbuf[r, pl.ds(64 - half + j * 16, 16)] = fz16

            pltpu.sync_copy(outbuf, acc_s.at[didx2.at[0]], add=True)
            pltpu.sync_copy(pbuf, den_s.at[didx8.at[0]], add=True)

            for g in range(C // 16):
                dv = didxr[b, pl.ds(g * 16, 16)]
                plsc.store_scatter(pbuf, [g * 16 + iota16, dv & 127], fz16)

        fetch(0, 0)
        fetch(1, 1)

        @pl.loop(0, CHUNKS_PER_W // 2 - 1)
        def _chunk(u):
            t0 = 2 * u
            process(0)
            fetch(t0 + 2, 0)
            process(1)
            fetch(t0 + 3, 1)

        process(0)
        process(1)

        plsc.subcore_barrier()

        @pl.loop(0, AR_SUB // 64)
        def _flush_acc(b):
            r0 = sid * AR_SUB + b * 64
            pltpu.sync_copy(acc_s.at[pl.ds(r0, 64)],
                            acc_out.at[cid].at[pl.ds(r0, 64)])

        @pl.when(sid < DROWS // 8)
        def _flush_den():
            pltpu.sync_copy(den_s.at[pl.ds(sid * 8, 8)],
                            den_out.at[cid].at[pl.ds(sid * 8, 8)])

    return sc_edge


def _make_sc_edge16():
    D = 16
    mesh = plsc.VectorSubcoreMesh(core_axis_name="c", subcore_axis_name="s")
    cp = pltpu.CompilerParams(
        needs_layout_passes=False, use_tc_tiling_on_sc=False
    )

    @functools.partial(
        pl.kernel,
        mesh=mesh,
        compiler_params=cp,
        out_type=[
            jax.ShapeDtypeStruct((2, NPAD, D), jnp.float32),
            jax.ShapeDtypeStruct((2, NPAD, 16), jnp.float32),
        ],
        scratch_types=[
            pltpu.VMEM((NPAD, 2), jnp.float32),
            pltpu.VMEM((2, C), jnp.int32),
            pltpu.VMEM((2, C), jnp.int32),
            pltpu.VMEM((2, C, D), jnp.float32),
            pltpu.VMEM((C, 16), jnp.float32),
            pltpu.VMEM((C, D), jnp.float32),
            pltpu.VMEM_SHARED((NPAD, D), jnp.float32),
            pltpu.VMEM_SHARED((NPAD, D), jnp.float32),
            pltpu.VMEM_SHARED((NPAD, 16), jnp.float32),
            pltpu.SemaphoreType.DMA,
            pltpu.SemaphoreType.DMA,
        ],
    )
    def sc_edge(h_hbm, ab_hbm, src_hbm, dst_hbm, acc_out, den_out,
                ab_t, sidx, didx, rowbuf, pbuf, zbuf, h_s, acc_s, den_s,
                sem0, sem1):
        cid = lax.axis_index("c")
        sid = lax.axis_index("s")
        wid = sid * 2 + cid
        zero16 = jnp.zeros((16,), jnp.int32)
        one16 = jnp.ones((16,), jnp.int32)
        iota16 = lax.iota(jnp.int32, 16)
        fz16 = jnp.zeros((16,), jnp.float32)
        sems = (sem0, sem1)
        base = wid * (CHUNKS_PER_W * C)

        @pl.loop(0, C)
        def _zero_bufs(r):
            zbuf[r, pl.ds(0, 16)] = fz16
            pbuf[r, pl.ds(0, 16)] = fz16

        base_row = sid * ROWS_PER_SUB
        pltpu.sync_copy(
            h_hbm.at[pl.ds(base_row, ROWS_PER_SUB)],
            h_s.at[pl.ds(base_row, ROWS_PER_SUB)],
        )

        @pl.loop(0, ROWS_PER_SUB // C)
        def _zero_spmem(b):
            r0 = base_row + b * C
            pltpu.sync_copy(zbuf, acc_s.at[pl.ds(r0, C)])
            pltpu.sync_copy(pbuf, den_s.at[pl.ds(r0, C)])

        pltpu.sync_copy(ab_hbm, ab_t)
        plsc.subcore_barrier()

        def fetch(t, b):
            off = base + t * C
            pltpu.sync_copy(src_hbm.at[pl.ds(off, C)], sidx.at[b])
            pltpu.sync_copy(dst_hbm.at[pl.ds(off, C)], didx.at[b])
            pltpu.async_copy(h_s.at[sidx.at[b]], rowbuf.at[b], sems[b])

        def process(b):
            pltpu.make_async_copy(
                h_s.at[sidx.at[b]], rowbuf.at[b], sems[b]).wait()
            for g in range(C // 16):
                sv = sidx[b, pl.ds(g * 16, 16)]
                dv = didx[b, pl.ds(g * 16, 16)]
                a_s = plsc.load_gather(ab_t, [sv, zero16])
                a_d = plsc.load_gather(ab_t, [dv, one16])
                e = a_s + a_d
                e = jnp.where(e >= 0.0, e, e * 0.2)
                p = jnp.exp(e)
                plsc.store_scatter(pbuf, [g * 16 + iota16, zero16], p)

            @pl.loop(0, C)
            def _scale(r):
                ps = pbuf[r, pl.ds(0, 16)][0]
                rowbuf[b, r, pl.ds(0, 16)] = rowbuf[b, r, pl.ds(0, 16)] * ps

            pltpu.sync_copy(rowbuf.at[b], acc_s.at[didx.at[b]], add=True)
            pltpu.sync_copy(pbuf, den_s.at[didx.at[b]], add=True)

        fetch(0, 0)
        fetch(1, 1)

        @pl.loop(0, CHUNKS_PER_W // 2 - 1)
        def _chunk(u):
            t0 = 2 * u
            process(0)
            fetch(t0 + 2, 0)
            process(1)
            fetch(t0 + 3, 1)

        process(0)
        process(1)

        plsc.subcore_barrier()

        @pl.loop(0, ROWS_PER_SUB // C)
        def _flush(b):
            r0 = base_row + b * C
            pltpu.sync_copy(acc_s.at[pl.ds(r0, C)], acc_out.at[cid].at[pl.ds(r0, C)])
            pltpu.sync_copy(den_s.at[pl.ds(r0, C)], den_out.at[cid].at[pl.ds(r0, C)])

    return sc_edge


_sc_edge_l1 = _make_sc_edge_wide()
_sc_edge_l2 = _make_sc_edge16()


def kernel(x, edge_index, W1, a_src1, a_dst1, b1, W2, a_src2, a_dst2, b2):
    loop = jnp.arange(N, dtype=edge_index.dtype)
    src = jnp.concatenate([edge_index[0], loop])
    dst = jnp.concatenate([edge_index[1], loop])
    src = jnp.pad(src, (0, E_PAD - E_TOT), constant_values=N)
    dst = jnp.pad(dst, (0, E_PAD - E_TOT), constant_values=N)
    xpad = jnp.pad(x, ((0, NPAD - N), (0, 0)))
    A1 = jnp.stack([a_src1, a_dst1], axis=1)
    A2 = jnp.stack([a_src2, a_dst2], axis=1)
    b2pad = jnp.pad(b2, (0, 16 - N_CLASSES))

    ht1, ab1 = _tc1(xpad, W1, A1)
    as1 = ab1[:, 0].reshape(NPAD // 128, 128)
    ad1 = ab1[:, 1].reshape(NPAD // 128, 128)
    acc1p, den1p = _sc_edge_l1(ht1, as1, ad1, src, dst)
    acc1 = acc1p.reshape(2, NPAD, HIDDEN)
    den1 = den1p.reshape(2, NPAD)[:, :, None]
    h2p, ab2 = _tc2(acc1, den1, b1, W2, A2)
    acc2, den2 = _sc_edge_l2(h2p, ab2, src, dst)
    out = _tc3(acc2, den2, b2pad)
    return out[:N, :N_CLASSES]


# speedup vs baseline: 184.6639x; 1.1510x over previous
import functools

import jax
import jax.numpy as jnp
from jax import lax
from jax.experimental import pallas as pl
from jax.experimental.pallas import tpu as pltpu
from jax.experimental.pallas import tpu_sc as plsc

N = 10000
NPAD = 10240
D_IN = 128
HIDDEN = 64
N_CLASSES = 8

NW = 32
C = 128
E_TOT = 640000 + N
CHUNKS_PER_W = 160
E_PAD = NW * C * CHUNKS_PER_W
ROWS_PER_SUB = NPAD // 16


def _tc1_body(x_ref, w_ref, a_ref, h_ref, ab_ref):
    h = jnp.dot(x_ref[...], w_ref[...], preferred_element_type=jnp.float32)
    h_ref[:, :HIDDEN] = h
    h_ref[:, HIDDEN:] = jnp.zeros((NPAD, 128 - HIDDEN), jnp.float32)
    ab_ref[...] = jnp.dot(h, a_ref[...], preferred_element_type=jnp.float32)


def _tc1(xpad, W1, A1):
    return pl.pallas_call(
        _tc1_body,
        out_shape=[
            jax.ShapeDtypeStruct((NPAD, 128), jnp.float32),
            jax.ShapeDtypeStruct((NPAD, 2), jnp.float32),
        ],
    )(xpad, W1, A1)


def _tc2_body(acc_ref, den_ref, b_ref, w_ref, a_ref, h2_ref, ab2_ref):
    num = acc_ref[0] + acc_ref[1]
    den = den_ref[0, :, 0:1] + den_ref[1, :, 0:1] + 1e-16
    out1 = jax.nn.relu(num / den + b_ref[...])
    h2 = jnp.dot(out1, w_ref[...], preferred_element_type=jnp.float32)
    h2_ref[...] = jnp.concatenate(
        [h2, jnp.zeros((NPAD, 16 - N_CLASSES), jnp.float32)], axis=1
    )
    ab2_ref[...] = jnp.dot(h2, a_ref[...], preferred_element_type=jnp.float32)


def _tc2(acc, den, b1, W2, A2):
    return pl.pallas_call(
        _tc2_body,
        out_shape=[
            jax.ShapeDtypeStruct((NPAD, 16), jnp.float32),
            jax.ShapeDtypeStruct((NPAD, 2), jnp.float32),
        ],
    )(acc, den, b1[None, :], W2, A2)


def _tc3_body(acc_ref, den_ref, b_ref, out_ref):
    den = den_ref[0, :, 0:1] + den_ref[1, :, 0:1] + 1e-16
    num = acc_ref[0] + acc_ref[1]
    out_ref[...] = num / den + b_ref[...]


def _tc3(acc, den, b2pad):
    return pl.pallas_call(
        _tc3_body,
        out_shape=jax.ShapeDtypeStruct((NPAD, 16), jnp.float32),
    )(acc, den, b2pad[None, :])


def _make_sc_edge_wide():
    mesh = plsc.VectorSubcoreMesh(core_axis_name="c", subcore_axis_name="s")
    cp = pltpu.CompilerParams(needs_layout_passes=False)
    AROWS = NPAD // 2
    AR_SUB = AROWS // 16

    @functools.partial(
        pl.kernel,
        mesh=mesh,
        compiler_params=cp,
        out_type=[
            jax.ShapeDtypeStruct((2, AROWS, 128), jnp.float32),
            jax.ShapeDtypeStruct((2, NPAD), jnp.float32),
        ],
        scratch_types=[
            pltpu.VMEM((NPAD // 128, 128), jnp.float32),
            pltpu.VMEM((NPAD // 128, 128), jnp.float32),
            pltpu.VMEM((2, C), jnp.int32),
            pltpu.VMEM((2, C), jnp.int32),
            pltpu.VMEM((2, C), jnp.int32),
            pltpu.VMEM((2, C + 16), jnp.int32),
            pltpu.VMEM((2, C, 128), jnp.float32),
            pltpu.VMEM((2, C, 128), jnp.float32),
            pltpu.VMEM((2, C + 16), jnp.float32),
            pltpu.VMEM((ROWS_PER_SUB,), jnp.float32),
            pltpu.VMEM_SHARED((AROWS, 128), jnp.float32),
            pltpu.VMEM_SHARED((NPAD,), jnp.float32),
            pltpu.SemaphoreType.DMA,
            pltpu.SemaphoreType.DMA,
            pltpu.SemaphoreType.DMA,
            pltpu.SemaphoreType.DMA,
            pltpu.SemaphoreType.DMA,
            pltpu.SemaphoreType.DMA,
        ],
    )
    def sc_edge(ht_hbm, as_hbm, ad_hbm, src_hbm, dst_hbm, acc_out, den_out,
                as_t, ad_t, sidx, didxr, didx2, ph, rowbuf, outbuf, pv, z1d,
                acc_s, den_s, sg0, sg1, ss0, ss1, sd0, sd1):
        cid = lax.axis_index("c")
        sid = lax.axis_index("s")
        wid = sid * 2 + cid
        iota16 = lax.iota(jnp.int32, 16)
        fz16 = jnp.zeros((16,), jnp.float32)
        sg = (sg0, sg1)
        ss = (ss0, ss1)
        sd = (sd0, sd1)
        base = wid * (CHUNKS_PER_W * C)

        @pl.loop(0, C)
        def _zero_bufs(r):
            for j in range(8):
                outbuf[0, r, pl.ds(j * 16, 16)] = fz16

        @pl.loop(0, ROWS_PER_SUB, step=16)
        def _zero_strip(i):
            z1d[pl.ds(i, 16)] = fz16

        @pl.loop(0, AR_SUB // 64)
        def _zero_acc(b):
            pltpu.sync_copy(
                outbuf.at[0].at[pl.ds(0, 64)],
                acc_s.at[pl.ds(sid * AR_SUB + b * 64, 64)],
            )

        pltpu.sync_copy(z1d, den_s.at[pl.ds(sid * ROWS_PER_SUB, ROWS_PER_SUB)])
        pltpu.sync_copy(as_hbm, as_t)
        pltpu.sync_copy(ad_hbm, ad_t)
        plsc.subcore_barrier()

        def fetch(t, b):
            off = base + t * C
            pltpu.sync_copy(src_hbm.at[pl.ds(off, C)], sidx.at[b])
            pltpu.sync_copy(dst_hbm.at[pl.ds(off, C)], didxr.at[b])
            pltpu.async_copy(ht_hbm.at[sidx.at[b]], rowbuf.at[b], sg[b])

        def wait_scatters(b):
            pltpu.make_async_copy(
                outbuf.at[b], acc_s.at[didx2.at[b]], ss[b]).wait()
            pltpu.make_async_copy(
                pv.at[b].at[pl.ds(0, C)], den_s.at[didxr.at[b]], sd[b]).wait()

        def process(b):
            pltpu.make_async_copy(
                ht_hbm.at[sidx.at[b]], rowbuf.at[b], sg[b]).wait()
            for g in range(C // 16):
                sv = sidx[b, pl.ds(g * 16, 16)]
                dv = didxr[b, pl.ds(g * 16, 16)]
                a_s = plsc.load_gather(
                    as_t, [lax.shift_right_logical(sv, 7), sv & 127])
                a_d = plsc.load_gather(
                    ad_t, [lax.shift_right_logical(dv, 7), dv & 127])
                e = a_s + a_d
                e = jnp.where(e >= 0.0, e, e * 0.2)
                p = jnp.exp(e)
                plsc.store_scatter(pv, [jnp.full((16,), b, jnp.int32),
                                        g * 16 + iota16], p)
                didx2[b, pl.ds(g * 16, 16)] = lax.shift_right_logical(dv, 1)
                plsc.store_scatter(ph, [jnp.full((16,), b, jnp.int32),
                                        g * 16 + iota16],
                                   lax.shift_left(dv & 1, 6))

            @pl.loop(0, C)
            def _scale(r):
                ps = pv[b, pl.ds(r, 16)][0]
                half = ph[b, pl.ds(r, 16)][0]
                for j in range(HIDDEN // 16):
                    outbuf[b, r, pl.ds(half + j * 16, 16)] = (
                        rowbuf[b, r, pl.ds(j * 16, 16)] * ps
                    )
                    outbuf[b, r, pl.ds(64 - half + j * 16, 16)] = fz16

            pltpu.async_copy(outbuf.at[b], acc_s.at[didx2.at[b]], ss[b],
                             add=True)
            pltpu.async_copy(pv.at[b].at[pl.ds(0, C)], den_s.at[didxr.at[b]],
                             sd[b], add=True)

        fetch(0, 0)
        fetch(1, 1)
        process(0)
        fetch(2, 0)
        process(1)
        fetch(3, 1)

        @pl.loop(1, CHUNKS_PER_W // 2 - 1)
        def _chunk(u):
            t0 = 2 * u
            wait_scatters(0)
            process(0)
            fetch(t0 + 2, 0)
            wait_scatters(1)
            process(1)
            fetch(t0 + 3, 1)

        wait_scatters(0)
        process(0)
        wait_scatters(1)
        process(1)
        wait_scatters(0)
        wait_scatters(1)

        plsc.subcore_barrier()

        @pl.loop(0, AR_SUB // 64)
        def _flush_acc(b):
            r0 = sid * AR_SUB + b * 64
            pltpu.sync_copy(acc_s.at[pl.ds(r0, 64)],
                            acc_out.at[cid].at[pl.ds(r0, 64)])

        pltpu.sync_copy(den_s.at[pl.ds(sid * ROWS_PER_SUB, ROWS_PER_SUB)],
                        den_out.at[cid].at[pl.ds(sid * ROWS_PER_SUB,
                                                 ROWS_PER_SUB)])

    return sc_edge


def _make_sc_edge16():
    D = 16
    mesh = plsc.VectorSubcoreMesh(core_axis_name="c", subcore_axis_name="s")
    cp = pltpu.CompilerParams(
        needs_layout_passes=False, use_tc_tiling_on_sc=False
    )

    @functools.partial(
        pl.kernel,
        mesh=mesh,
        compiler_params=cp,
        out_type=[
            jax.ShapeDtypeStruct((2, NPAD, D), jnp.float32),
            jax.ShapeDtypeStruct((2, NPAD), jnp.float32),
        ],
        scratch_types=[
            pltpu.VMEM((NPAD, 2), jnp.float32),
            pltpu.VMEM((2, C), jnp.int32),
            pltpu.VMEM((2, C), jnp.int32),
            pltpu.VMEM((2, C, D), jnp.float32),
            pltpu.VMEM((2, C, D), jnp.float32),
            pltpu.VMEM((2, C + 16), jnp.float32),
            pltpu.VMEM((C, D), jnp.float32),
            pltpu.VMEM_SHARED((NPAD, D), jnp.float32),
            pltpu.VMEM_SHARED((NPAD, D), jnp.float32),
            pltpu.VMEM_SHARED((NPAD,), jnp.float32),
            pltpu.SemaphoreType.DMA,
            pltpu.SemaphoreType.DMA,
            pltpu.SemaphoreType.DMA,
            pltpu.SemaphoreType.DMA,
            pltpu.SemaphoreType.DMA,
            pltpu.SemaphoreType.DMA,
        ],
    )
    def sc_edge(h_hbm, ab_hbm, src_hbm, dst_hbm, acc_out, den_out,
                ab_t, sidx, didx, rowbuf, outbuf, pv, zbuf, h_s, acc_s, den_s,
                sg0, sg1, ss0, ss1, sd0, sd1):
        cid = lax.axis_index("c")
        sid = lax.axis_index("s")
        wid = sid * 2 + cid
        zero16 = jnp.zeros((16,), jnp.int32)
        one16 = jnp.ones((16,), jnp.int32)
        iota16 = lax.iota(jnp.int32, 16)
        fz16 = jnp.zeros((16,), jnp.float32)
        sg = (sg0, sg1)
        ss = (ss0, ss1)
        sd = (sd0, sd1)
        base = wid * (CHUNKS_PER_W * C)

        @pl.loop(0, C)
        def _zero_bufs(r):
            zbuf[r, pl.ds(0, 16)] = fz16

        @pl.loop(0, C + 16, step=16)
        def _zero_pv(i):
            pv[0, pl.ds(i, 16)] = fz16

        base_row = sid * ROWS_PER_SUB
        pltpu.sync_copy(
            h_hbm.at[pl.ds(base_row, ROWS_PER_SUB)],
            h_s.at[pl.ds(base_row, ROWS_PER_SUB)],
        )

        @pl.loop(0, ROWS_PER_SUB // C)
        def _zero_spmem(b):
            r0 = base_row + b * C
            pltpu.sync_copy(zbuf, acc_s.at[pl.ds(r0, C)])

        @pl.loop(0, ROWS_PER_SUB // C)
        def _zero_den(b):
            pltpu.sync_copy(pv.at[0].at[pl.ds(0, C)],
                            den_s.at[pl.ds(base_row + b * C, C)])

        pltpu.sync_copy(ab_hbm, ab_t)
        plsc.subcore_barrier()

        def fetch(t, b):
            off = base + t * C
            pltpu.sync_copy(src_hbm.at[pl.ds(off, C)], sidx.at[b])
            pltpu.sync_copy(dst_hbm.at[pl.ds(off, C)], didx.at[b])
            pltpu.async_copy(h_s.at[sidx.at[b]], rowbuf.at[b], sg[b])

        def wait_scatters(b):
            pltpu.make_async_copy(
                outbuf.at[b], acc_s.at[didx.at[b]], ss[b]).wait()
            pltpu.make_async_copy(
                pv.at[b].at[pl.ds(0, C)], den_s.at[didx.at[b]], sd[b]).wait()

        def process(b):
            pltpu.make_async_copy(
                h_s.at[sidx.at[b]], rowbuf.at[b], sg[b]).wait()
            for g in range(C // 16):
                sv = sidx[b, pl.ds(g * 16, 16)]
                dv = didx[b, pl.ds(g * 16, 16)]
                a_s = plsc.load_gather(ab_t, [sv, zero16])
                a_d = plsc.load_gather(ab_t, [dv, one16])
                e = a_s + a_d
                e = jnp.where(e >= 0.0, e, e * 0.2)
                p = jnp.exp(e)
                plsc.store_scatter(pv, [jnp.full((16,), b, jnp.int32),
                                        g * 16 + iota16], p)

            @pl.loop(0, C)
            def _scale(r):
                ps = pv[b, pl.ds(r, 16)][0]
                outbuf[b, r, pl.ds(0, 16)] = rowbuf[b, r, pl.ds(0, 16)] * ps

            pltpu.async_copy(outbuf.at[b], acc_s.at[didx.at[b]], ss[b],
                             add=True)
            pltpu.async_copy(pv.at[b].at[pl.ds(0, C)], den_s.at[didx.at[b]],
                             sd[b], add=True)

        fetch(0, 0)
        fetch(1, 1)
        process(0)
        fetch(2, 0)
        process(1)
        fetch(3, 1)

        @pl.loop(1, CHUNKS_PER_W // 2 - 1)
        def _chunk(u):
            t0 = 2 * u
            wait_scatters(0)
            process(0)
            fetch(t0 + 2, 0)
            wait_scatters(1)
            process(1)
            fetch(t0 + 3, 1)

        wait_scatters(0)
        process(0)
        wait_scatters(1)
        process(1)
        wait_scatters(0)
        wait_scatters(1)

        plsc.subcore_barrier()

        @pl.loop(0, ROWS_PER_SUB // C)
        def _flush(b):
            r0 = base_row + b * C
            pltpu.sync_copy(acc_s.at[pl.ds(r0, C)], acc_out.at[cid].at[pl.ds(r0, C)])

        pltpu.sync_copy(den_s.at[pl.ds(base_row, ROWS_PER_SUB)],
                        den_out.at[cid].at[pl.ds(base_row, ROWS_PER_SUB)])

    return sc_edge


_sc_edge_l1 = _make_sc_edge_wide()
_sc_edge_l2 = _make_sc_edge16()


def kernel(x, edge_index, W1, a_src1, a_dst1, b1, W2, a_src2, a_dst2, b2):
    loop = jnp.arange(N, dtype=edge_index.dtype)
    src = jnp.concatenate([edge_index[0], loop])
    dst = jnp.concatenate([edge_index[1], loop])
    src = jnp.pad(src, (0, E_PAD - E_TOT), constant_values=N)
    dst = jnp.pad(dst, (0, E_PAD - E_TOT), constant_values=N)
    xpad = jnp.pad(x, ((0, NPAD - N), (0, 0)))
    A1 = jnp.stack([a_src1, a_dst1], axis=1)
    A2 = jnp.stack([a_src2, a_dst2], axis=1)
    b2pad = jnp.pad(b2, (0, 16 - N_CLASSES))

    ht1, ab1 = _tc1(xpad, W1, A1)
    as1 = ab1[:, 0].reshape(NPAD // 128, 128)
    ad1 = ab1[:, 1].reshape(NPAD // 128, 128)
    acc1p, den1p = _sc_edge_l1(ht1, as1, ad1, src, dst)
    acc1 = acc1p.reshape(2, NPAD, HIDDEN)
    den1 = den1p[:, :, None]
    h2p, ab2 = _tc2(acc1, den1, b1, W2, A2)
    acc2, den2 = _sc_edge_l2(h2p, ab2, src, dst)
    out = _tc3(acc2, den2[:, :, None], b2pad)
    return out[:N, :N_CLASSES]


# speedup vs baseline: 192.6569x; 1.0433x over previous
import functools

import jax
import jax.numpy as jnp
from jax import lax
from jax.experimental import pallas as pl
from jax.experimental.pallas import tpu as pltpu
from jax.experimental.pallas import tpu_sc as plsc

N = 10000
NPAD = 10240
D_IN = 128
HIDDEN = 64
N_CLASSES = 8

NW = 32
C = 128
E_TOT = 640000 + N
CHUNKS_PER_W = 160
E_PAD = NW * C * CHUNKS_PER_W
ROWS_PER_SUB = NPAD // 16


def _tc1_body(x_ref, w_ref, a_ref, h_ref, ab_ref):
    h = jnp.dot(x_ref[...], w_ref[...], preferred_element_type=jnp.float32)
    h_ref[...] = h
    ab_ref[...] = jnp.dot(h, a_ref[...], preferred_element_type=jnp.float32)


def _tc1(xpad, W1, A1):
    return pl.pallas_call(
        _tc1_body,
        out_shape=[
            jax.ShapeDtypeStruct((NPAD, HIDDEN), jnp.float32),
            jax.ShapeDtypeStruct((NPAD, 2), jnp.float32),
        ],
    )(xpad, W1, A1)


def _tc2_body(acc_ref, den_ref, b_ref, w_ref, a_ref, h2_ref, ab2_ref):
    num = acc_ref[0] + acc_ref[1]
    den = den_ref[0, :, 0:1] + den_ref[1, :, 0:1] + 1e-16
    out1 = jax.nn.relu(num / den + b_ref[...])
    h2 = jnp.dot(out1, w_ref[...], preferred_element_type=jnp.float32)
    h2_ref[...] = jnp.concatenate(
        [h2, jnp.zeros((NPAD, 16 - N_CLASSES), jnp.float32)], axis=1
    )
    ab2_ref[...] = jnp.dot(h2, a_ref[...], preferred_element_type=jnp.float32)


def _tc2(acc, den, b1, W2, A2):
    return pl.pallas_call(
        _tc2_body,
        out_shape=[
            jax.ShapeDtypeStruct((NPAD, 16), jnp.float32),
            jax.ShapeDtypeStruct((NPAD, 2), jnp.float32),
        ],
    )(acc, den, b1[None, :], W2, A2)


def _tc3_body(acc_ref, den_ref, b_ref, out_ref):
    den = den_ref[0, :, 0:1] + den_ref[1, :, 0:1] + 1e-16
    num = acc_ref[0] + acc_ref[1]
    out_ref[...] = num / den + b_ref[...]


def _tc3(acc, den, b2pad):
    return pl.pallas_call(
        _tc3_body,
        out_shape=jax.ShapeDtypeStruct((NPAD, 16), jnp.float32),
    )(acc, den, b2pad[None, :])


def _make_sc_edge_wide():
    mesh = plsc.VectorSubcoreMesh(core_axis_name="c", subcore_axis_name="s")
    cp = pltpu.CompilerParams(
        needs_layout_passes=False, use_tc_tiling_on_sc=False
    )

    @functools.partial(
        pl.kernel,
        mesh=mesh,
        compiler_params=cp,
        out_type=[
            jax.ShapeDtypeStruct((2, NPAD, HIDDEN), jnp.float32),
            jax.ShapeDtypeStruct((2, NPAD), jnp.float32),
        ],
        scratch_types=[
            pltpu.VMEM((NPAD // 128, 128), jnp.float32),
            pltpu.VMEM((NPAD // 128, 128), jnp.float32),
            pltpu.VMEM((2, C), jnp.int32),
            pltpu.VMEM((2, C), jnp.int32),
            pltpu.VMEM((2, C), jnp.int32),
            pltpu.VMEM((2, C + 16), jnp.int32),
            pltpu.VMEM((2, C, 128), jnp.float32),
            pltpu.VMEM((2, C, HIDDEN), jnp.float32),
            pltpu.VMEM((2, C + 16), jnp.float32),
            pltpu.VMEM((ROWS_PER_SUB,), jnp.float32),
            pltpu.VMEM_SHARED((NPAD, HIDDEN), jnp.float32),
            pltpu.VMEM_SHARED((NPAD,), jnp.float32),
            pltpu.SemaphoreType.DMA,
            pltpu.SemaphoreType.DMA,
            pltpu.SemaphoreType.DMA,
            pltpu.SemaphoreType.DMA,
            pltpu.SemaphoreType.DMA,
            pltpu.SemaphoreType.DMA,
        ],
    )
    def sc_edge(ht_hbm, as_hbm, ad_hbm, src_hbm, dst_hbm, acc_out, den_out,
                as_t, ad_t, sidx, didxr, gidx, ph, rowbuf, outbuf, pv, z1d,
                acc_s, den_s, sg0, sg1, ss0, ss1, sd0, sd1):
        cid = lax.axis_index("c")
        sid = lax.axis_index("s")
        wid = sid * 2 + cid
        iota16 = lax.iota(jnp.int32, 16)
        fz16 = jnp.zeros((16,), jnp.float32)
        sg = (sg0, sg1)
        ss = (ss0, ss1)
        sd = (sd0, sd1)
        base = wid * (CHUNKS_PER_W * C)

        @pl.loop(0, C)
        def _zero_bufs(r):
            for j in range(HIDDEN // 16):
                outbuf[0, r, pl.ds(j * 16, 16)] = fz16

        @pl.loop(0, ROWS_PER_SUB, step=16)
        def _zero_strip(i):
            z1d[pl.ds(i, 16)] = fz16

        @pl.loop(0, ROWS_PER_SUB // 64)
        def _zero_acc(b):
            pltpu.sync_copy(
                outbuf.at[0].at[pl.ds(0, 64)],
                acc_s.at[pl.ds(sid * ROWS_PER_SUB + b * 64, 64)],
            )

        pltpu.sync_copy(z1d, den_s.at[pl.ds(sid * ROWS_PER_SUB, ROWS_PER_SUB)])
        pltpu.sync_copy(as_hbm, as_t)
        pltpu.sync_copy(ad_hbm, ad_t)
        plsc.subcore_barrier()

        def fetch(t, b):
            off = base + t * C
            pltpu.sync_copy(src_hbm.at[pl.ds(off, C)], sidx.at[b])
            pltpu.sync_copy(dst_hbm.at[pl.ds(off, C)], didxr.at[b])
            for g in range(C // 16):
                gidx[b, pl.ds(g * 16, 16)] = lax.shift_right_logical(
                    sidx[b, pl.ds(g * 16, 16)], 1)
            pltpu.async_copy(ht_hbm.at[gidx.at[b]], rowbuf.at[b], sg[b])

        def wait_scatters(b):
            pltpu.make_async_copy(
                outbuf.at[b], acc_s.at[didxr.at[b]], ss[b]).wait()
            pltpu.make_async_copy(
                pv.at[b].at[pl.ds(0, C)], den_s.at[didxr.at[b]], sd[b]).wait()

        def process(b):
            pltpu.make_async_copy(
                ht_hbm.at[gidx.at[b]], rowbuf.at[b], sg[b]).wait()
            for g in range(C // 16):
                sv = sidx[b, pl.ds(g * 16, 16)]
                dv = didxr[b, pl.ds(g * 16, 16)]
                a_s = plsc.load_gather(
                    as_t, [lax.shift_right_logical(sv, 7), sv & 127])
                a_d = plsc.load_gather(
                    ad_t, [lax.shift_right_logical(dv, 7), dv & 127])
                e = a_s + a_d
                e = jnp.where(e >= 0.0, e, e * 0.2)
                p = jnp.exp(e)
                plsc.store_scatter(pv, [jnp.full((16,), b, jnp.int32),
                                        g * 16 + iota16], p)
                plsc.store_scatter(ph, [jnp.full((16,), b, jnp.int32),
                                        g * 16 + iota16],
                                   lax.shift_left(sv & 1, 6))

            @pl.loop(0, C)
            def _scale(r):
                ps = pv[b, pl.ds(r, 16)][0]
                half = ph[b, pl.ds(r, 16)][0]
                for j in range(HIDDEN // 16):
                    outbuf[b, r, pl.ds(j * 16, 16)] = (
                        rowbuf[b, r, pl.ds(half + j * 16, 16)] * ps
                    )

            pltpu.async_copy(outbuf.at[b], acc_s.at[didxr.at[b]], ss[b],
                             add=True)
            pltpu.async_copy(pv.at[b].at[pl.ds(0, C)], den_s.at[didxr.at[b]],
                             sd[b], add=True)

        fetch(0, 0)
        fetch(1, 1)
        process(0)
        fetch(2, 0)
        process(1)
        fetch(3, 1)

        @pl.loop(1, CHUNKS_PER_W // 2 - 1)
        def _chunk(u):
            t0 = 2 * u
            wait_scatters(0)
            process(0)
            fetch(t0 + 2, 0)
            wait_scatters(1)
            process(1)
            fetch(t0 + 3, 1)

        wait_scatters(0)
        process(0)
        wait_scatters(1)
        process(1)
        wait_scatters(0)
        wait_scatters(1)

        plsc.subcore_barrier()

        @pl.loop(0, ROWS_PER_SUB // 64)
        def _flush_acc(b):
            r0 = sid * ROWS_PER_SUB + b * 64
            pltpu.sync_copy(acc_s.at[pl.ds(r0, 64)],
                            acc_out.at[cid].at[pl.ds(r0, 64)])

        pltpu.sync_copy(den_s.at[pl.ds(sid * ROWS_PER_SUB, ROWS_PER_SUB)],
                        den_out.at[cid].at[pl.ds(sid * ROWS_PER_SUB,
                                                 ROWS_PER_SUB)])

    return sc_edge


def _make_sc_edge16():
    D = 16
    mesh = plsc.VectorSubcoreMesh(core_axis_name="c", subcore_axis_name="s")
    cp = pltpu.CompilerParams(
        needs_layout_passes=False, use_tc_tiling_on_sc=False
    )

    @functools.partial(
        pl.kernel,
        mesh=mesh,
        compiler_params=cp,
        out_type=[
            jax.ShapeDtypeStruct((2, NPAD, D), jnp.float32),
            jax.ShapeDtypeStruct((2, NPAD), jnp.float32),
        ],
        scratch_types=[
            pltpu.VMEM((NPAD, 2), jnp.float32),
            pltpu.VMEM((2, C), jnp.int32),
            pltpu.VMEM((2, C), jnp.int32),
            pltpu.VMEM((2, C, D), jnp.float32),
            pltpu.VMEM((2, C, D), jnp.float32),
            pltpu.VMEM((2, C + 16), jnp.float32),
            pltpu.VMEM((C, D), jnp.float32),
            pltpu.VMEM_SHARED((NPAD, D), jnp.float32),
            pltpu.VMEM_SHARED((NPAD, D), jnp.float32),
            pltpu.VMEM_SHARED((NPAD,), jnp.float32),
            pltpu.SemaphoreType.DMA,
            pltpu.SemaphoreType.DMA,
            pltpu.SemaphoreType.DMA,
            pltpu.SemaphoreType.DMA,
            pltpu.SemaphoreType.DMA,
            pltpu.SemaphoreType.DMA,
        ],
    )
    def sc_edge(h_hbm, ab_hbm, src_hbm, dst_hbm, acc_out, den_out,
                ab_t, sidx, didx, rowbuf, outbuf, pv, zbuf, h_s, acc_s, den_s,
                sg0, sg1, ss0, ss1, sd0, sd1):
        cid = lax.axis_index("c")
        sid = lax.axis_index("s")
        wid = sid * 2 + cid
        zero16 = jnp.zeros((16,), jnp.int32)
        one16 = jnp.ones((16,), jnp.int32)
        iota16 = lax.iota(jnp.int32, 16)
        fz16 = jnp.zeros((16,), jnp.float32)
        sg = (sg0, sg1)
        ss = (ss0, ss1)
        sd = (sd0, sd1)
        base = wid * (CHUNKS_PER_W * C)

        @pl.loop(0, C)
        def _zero_bufs(r):
            zbuf[r, pl.ds(0, 16)] = fz16

        @pl.loop(0, C + 16, step=16)
        def _zero_pv(i):
            pv[0, pl.ds(i, 16)] = fz16

        base_row = sid * ROWS_PER_SUB
        pltpu.sync_copy(
            h_hbm.at[pl.ds(base_row, ROWS_PER_SUB)],
            h_s.at[pl.ds(base_row, ROWS_PER_SUB)],
        )

        @pl.loop(0, ROWS_PER_SUB // C)
        def _zero_spmem(b):
            r0 = base_row + b * C
            pltpu.sync_copy(zbuf, acc_s.at[pl.ds(r0, C)])

        @pl.loop(0, ROWS_PER_SUB // C)
        def _zero_den(b):
            pltpu.sync_copy(pv.at[0].at[pl.ds(0, C)],
                            den_s.at[pl.ds(base_row + b * C, C)])

        pltpu.sync_copy(ab_hbm, ab_t)
        plsc.subcore_barrier()

        def fetch(t, b):
            off = base + t * C
            pltpu.sync_copy(src_hbm.at[pl.ds(off, C)], sidx.at[b])
            pltpu.sync_copy(dst_hbm.at[pl.ds(off, C)], didx.at[b])
            pltpu.async_copy(h_s.at[sidx.at[b]], rowbuf.at[b], sg[b])

        def wait_scatters(b):
            pltpu.make_async_copy(
                outbuf.at[b], acc_s.at[didx.at[b]], ss[b]).wait()
            pltpu.make_async_copy(
                pv.at[b].at[pl.ds(0, C)], den_s.at[didx.at[b]], sd[b]).wait()

        def process(b):
            pltpu.make_async_copy(
                h_s.at[sidx.at[b]], rowbuf.at[b], sg[b]).wait()
            for g in range(C // 16):
                sv = sidx[b, pl.ds(g * 16, 16)]
                dv = didx[b, pl.ds(g * 16, 16)]
                a_s = plsc.load_gather(ab_t, [sv, zero16])
                a_d = plsc.load_gather(ab_t, [dv, one16])
                e = a_s + a_d
                e = jnp.where(e >= 0.0, e, e * 0.2)
                p = jnp.exp(e)
                plsc.store_scatter(pv, [jnp.full((16,), b, jnp.int32),
                                        g * 16 + iota16], p)

            @pl.loop(0, C)
            def _scale(r):
                ps = pv[b, pl.ds(r, 16)][0]
                outbuf[b, r, pl.ds(0, 16)] = rowbuf[b, r, pl.ds(0, 16)] * ps

            pltpu.async_copy(outbuf.at[b], acc_s.at[didx.at[b]], ss[b],
                             add=True)
            pltpu.async_copy(pv.at[b].at[pl.ds(0, C)], den_s.at[didx.at[b]],
                             sd[b], add=True)

        fetch(0, 0)
        fetch(1, 1)
        process(0)
        fetch(2, 0)
        process(1)
        fetch(3, 1)

        @pl.loop(1, CHUNKS_PER_W // 2 - 1)
        def _chunk(u):
            t0 = 2 * u
            wait_scatters(0)
            process(0)
            fetch(t0 + 2, 0)
            wait_scatters(1)
            process(1)
            fetch(t0 + 3, 1)

        wait_scatters(0)
        process(0)
        wait_scatters(1)
        process(1)
        wait_scatters(0)
        wait_scatters(1)

        plsc.subcore_barrier()

        @pl.loop(0, ROWS_PER_SUB // C)
        def _flush(b):
            r0 = base_row + b * C
            pltpu.sync_copy(acc_s.at[pl.ds(r0, C)], acc_out.at[cid].at[pl.ds(r0, C)])

        pltpu.sync_copy(den_s.at[pl.ds(base_row, ROWS_PER_SUB)],
                        den_out.at[cid].at[pl.ds(base_row, ROWS_PER_SUB)])

    return sc_edge


_sc_edge_l1 = _make_sc_edge_wide()
_sc_edge_l2 = _make_sc_edge16()


def kernel(x, edge_index, W1, a_src1, a_dst1, b1, W2, a_src2, a_dst2, b2):
    loop = jnp.arange(N, dtype=edge_index.dtype)
    src = jnp.concatenate([edge_index[0], loop])
    dst = jnp.concatenate([edge_index[1], loop])
    src = jnp.pad(src, (0, E_PAD - E_TOT), constant_values=N)
    dst = jnp.pad(dst, (0, E_PAD - E_TOT), constant_values=N)
    xpad = jnp.pad(x, ((0, NPAD - N), (0, 0)))
    A1 = jnp.stack([a_src1, a_dst1], axis=1)
    A2 = jnp.stack([a_src2, a_dst2], axis=1)
    b2pad = jnp.pad(b2, (0, 16 - N_CLASSES))

    h1, ab1 = _tc1(xpad, W1, A1)
    as1 = ab1[:, 0].reshape(NPAD // 128, 128)
    ad1 = ab1[:, 1].reshape(NPAD // 128, 128)
    acc1, den1p = _sc_edge_l1(h1.reshape(NPAD // 2, 128), as1, ad1, src, dst)
    den1 = den1p[:, :, None]
    h2p, ab2 = _tc2(acc1, den1, b1, W2, A2)
    acc2, den2 = _sc_edge_l2(h2p, ab2, src, dst)
    out = _tc3(acc2, den2[:, :, None], b2pad)
    return out[:N, :N_CLASSES]
